# baseline (speedup 1.0000x reference)
import functools

import jax
import jax.numpy as jnp
from jax import lax
from jax.experimental import pallas as pl
from jax.experimental.pallas import tpu as pltpu
from jax.experimental.pallas import tpu_sc as plsc

TYPE_NUM = 1000000
TYPE_DIM = 64
BATCH = 16384

_INFO = plsc.get_sparse_core_info()
_NC = _INFO.num_cores
_NS = _INFO.num_subcores
_NW = _NC * _NS
_CHUNK = 128
_TOTAL = BATCH * 2
_ROWS = _TOTAL // _CHUNK
_RPW = _ROWS // _NW


def _make_gather():
    mesh = plsc.VectorSubcoreMesh(core_axis_name="c", subcore_axis_name="s")

    @functools.partial(
        pl.kernel,
        mesh=mesh,
        out_type=jax.ShapeDtypeStruct((_ROWS, _CHUNK, 2 * TYPE_DIM), jnp.float32),
        scratch_types=[
            pltpu.VMEM((_RPW, _CHUNK), jnp.int32),
            pltpu.VMEM((_RPW // 2, _CHUNK, 2 * TYPE_DIM), jnp.float32),
            pltpu.SemaphoreType.DMA,
        ],
    )
    def gather_kernel(table_hbm, idx_hbm, out_hbm, idx_v, rows_v, sem):
        wid = lax.axis_index("s") * _NC + lax.axis_index("c")
        base = wid * _RPW
        half = _RPW // 2
        pltpu.sync_copy(idx_hbm.at[pl.ds(base, _RPW)], idx_v)
        for p in range(2):
            copies = [
                pltpu.async_copy(
                    table_hbm.at[idx_v.at[p * half + j]], rows_v.at[j], sem
                )
                for j in range(half)
            ]
            for c in copies:
                c.wait()
            pltpu.sync_copy(rows_v, out_hbm.at[pl.ds(base + p * half, half)])

    return gather_kernel


_GATHER = _make_gather()


def kernel(inputs, type_matrix):
    idx = jnp.reshape(inputs.astype(jnp.int32), (-1,))
    table2 = jnp.reshape(type_matrix, (TYPE_NUM // 2, 2 * TYPE_DIM))
    pairs = _GATHER(table2, jnp.reshape(idx >> 1, (_ROWS, _CHUNK)))
    pairs = jnp.reshape(pairs, (_TOTAL, 2 * TYPE_DIM))
    half = jnp.where(
        (idx & 1)[:, None] == 1, pairs[:, TYPE_DIM:], pairs[:, :TYPE_DIM]
    )
    return jnp.reshape(half, (BATCH, 2 * TYPE_DIM))


# speedup vs baseline: 1.6159x; 1.6159x over previous
import functools

import jax
import jax.numpy as jnp
from jax import lax
from jax.experimental import pallas as pl
from jax.experimental.pallas import tpu as pltpu
from jax.experimental.pallas import tpu_sc as plsc

TYPE_NUM = 1000000
TYPE_DIM = 64
BATCH = 16384
_SPLIT = 499968
_HALF = TYPE_NUM // 2

_INFO = plsc.get_sparse_core_info()
_NC = _INFO.num_cores
_NS = _INFO.num_subcores
_NW = _NC * _NS
_TOTAL = BATCH * 2
_IPW = _TOTAL // _NW
_CHUNK = 128
_NCHUNK = _IPW // _CHUNK

_BLK = 2688
_GRID = _SPLIT // _BLK


def _repack_kernel(src_lo, src_hi, dst):
    dst[:, :TYPE_DIM] = jnp.transpose(src_lo[...])
    dst[:, TYPE_DIM:] = jnp.transpose(src_hi[...])


_repack = pl.pallas_call(
    _repack_kernel,
    grid=(_GRID,),
    in_specs=[
        pl.BlockSpec((TYPE_DIM, _BLK), lambda j: (0, j)),
        pl.BlockSpec((TYPE_DIM, _BLK), lambda j: (0, j + _GRID)),
    ],
    out_specs=pl.BlockSpec((_BLK, 2 * TYPE_DIM), lambda j: (j, 0)),
    out_shape=jax.ShapeDtypeStruct((_HALF, 2 * TYPE_DIM), jnp.float32),
)


def _tail_kernel(packed_ref, tsrc_ref, out_ref):
    tt = jnp.transpose(tsrc_ref[...])
    out_ref[:, :TYPE_DIM] = tt[:32, :]
    out_ref[:, TYPE_DIM:] = tt[32:, :]


_tail = pl.pallas_call(
    _tail_kernel,
    grid=(1,),
    in_specs=[
        pl.BlockSpec(memory_space=pl.ANY),
        pl.BlockSpec((TYPE_DIM, TYPE_DIM), lambda j: (0, 0)),
    ],
    out_specs=pl.BlockSpec((32, 2 * TYPE_DIM), lambda j: (_SPLIT // 32, 0)),
    out_shape=jax.ShapeDtypeStruct((_HALF, 2 * TYPE_DIM), jnp.float32),
    input_output_aliases={0: 0},
)


def _make_gather():
    mesh = plsc.VectorSubcoreMesh(core_axis_name="c", subcore_axis_name="s")

    @functools.partial(
        pl.kernel,
        mesh=mesh,
        compiler_params=pltpu.CompilerParams(needs_layout_passes=False),
        out_type=jax.ShapeDtypeStruct((BATCH, 2 * TYPE_DIM), jnp.float32),
        scratch_types=[
            pltpu.VMEM((_NCHUNK, _CHUNK), jnp.int32),
            pltpu.VMEM((_NCHUNK, _CHUNK), jnp.int32),
            pltpu.VMEM((_NCHUNK, _CHUNK), jnp.int32),
            pltpu.VMEM((_CHUNK, 2 * TYPE_DIM), jnp.float32),
            pltpu.VMEM((_CHUNK // 2, 2 * TYPE_DIM), jnp.float32),
            pltpu.SemaphoreType.DMA,
        ],
    )
    def gather_kernel(table_hbm, idx_hbm, out_hbm, idx_v, pid_v, cb_v,
                      pair_v, out_v, sem):
        wid = lax.axis_index("s") * _NC + lax.axis_index("c")
        ibase = wid * _NCHUNK
        obase = wid * (_IPW // 2)
        pltpu.sync_copy(idx_hbm.at[pl.ds(ibase, _NCHUNK)], idx_v)
        for j in range(_NCHUNK):
            for m in range(_CHUNK // 16):
                v = idx_v[j, pl.ds(m * 16, 16)]
                big = jnp.where(v >= _SPLIT, jnp.int32(_SPLIT), jnp.int32(0))
                wrap = jnp.where(v >= 2 * _SPLIT + 32, jnp.int32(32),
                                 jnp.int32(0))
                pid_v[j, pl.ds(m * 16, 16)] = v - big - wrap
                hi = jnp.where(
                    (v >= _SPLIT) & (v < 2 * _SPLIT), jnp.int32(TYPE_DIM),
                    jnp.int32(0))
                hi2 = jnp.where(v >= 2 * _SPLIT + 32, jnp.int32(TYPE_DIM),
                                jnp.int32(0))
                cb_v[j, pl.ds(m * 16, 16)] = hi + hi2

        def body(ch, _):
            pltpu.async_copy(
                table_hbm.at[pid_v.at[ch]], pair_v, sem).wait()
            for kg in range(_CHUNK // 16):
                rows16 = lax.iota(jnp.int32, 16) + kg * 16
                orow16 = rows16 >> 1
                ocb16 = (rows16 & 1) * TYPE_DIM
                cb16 = cb_v[ch, pl.ds(kg * 16, 16)]
                for col in range(TYPE_DIM):
                    vals = plsc.load_gather(pair_v, [rows16, cb16 + col])
                    plsc.store_scatter(out_v, [orow16, ocb16 + col], vals)
            pltpu.sync_copy(
                out_v,
                out_hbm.at[pl.ds(obase + ch * (_CHUNK // 2), _CHUNK // 2)])
            return ()

        lax.fori_loop(0, _NCHUNK, body, (), unroll=False)

    return gather_kernel


_GATHER = _make_gather()


def kernel(inputs, type_matrix):
    table_t = jnp.transpose(type_matrix)
    packed = _repack(table_t, table_t)
    tail_src = lax.slice(table_t, (0, 2 * _SPLIT), (TYPE_DIM, TYPE_NUM))
    packed = _tail(packed, tail_src)
    idx = jnp.reshape(inputs.astype(jnp.int32), (_TOTAL // _CHUNK, _CHUNK))
    return _GATHER(packed, idx)


# speedup vs baseline: 2.5547x; 1.5810x over previous
import functools

import jax
import jax.numpy as jnp
from jax import lax
from jax.experimental import pallas as pl
from jax.experimental.pallas import tpu as pltpu
from jax.experimental.pallas import tpu_sc as plsc

TYPE_NUM = 1000000
TYPE_DIM = 64
BATCH = 16384
_SPLIT = 499968
_HALF = TYPE_NUM // 2

_INFO = plsc.get_sparse_core_info()
_NC = _INFO.num_cores
_NS = _INFO.num_subcores
_NW = _NC * _NS
_TOTAL = BATCH * 2
_IPW = _TOTAL // _NW
_CHUNK = 128
_NCHUNK = _IPW // _CHUNK

_BLK = 8064
_GRID = _SPLIT // _BLK


def _repack_kernel(src_lo, src_hi, dst):
    dst[:, :TYPE_DIM] = jnp.transpose(src_lo[...])
    dst[:, TYPE_DIM:] = jnp.transpose(src_hi[...])


_repack = pl.pallas_call(
    _repack_kernel,
    grid=(_GRID,),
    in_specs=[
        pl.BlockSpec((TYPE_DIM, _BLK), lambda j: (0, j)),
        pl.BlockSpec((TYPE_DIM, _BLK), lambda j: (0, j + _GRID)),
    ],
    out_specs=pl.BlockSpec((_BLK, 2 * TYPE_DIM), lambda j: (j, 0)),
    out_shape=jax.ShapeDtypeStruct((_HALF, 2 * TYPE_DIM), jnp.float32),
)


def _tail_kernel(packed_ref, tsrc_ref, out_ref):
    tt = jnp.transpose(tsrc_ref[...])
    out_ref[:, :TYPE_DIM] = tt[:32, :]
    out_ref[:, TYPE_DIM:] = tt[32:, :]


_tail = pl.pallas_call(
    _tail_kernel,
    grid=(1,),
    in_specs=[
        pl.BlockSpec(memory_space=pl.ANY),
        pl.BlockSpec((TYPE_DIM, TYPE_DIM), lambda j: (0, 0)),
    ],
    out_specs=pl.BlockSpec((32, 2 * TYPE_DIM), lambda j: (_SPLIT // 32, 0)),
    out_shape=jax.ShapeDtypeStruct((_HALF, 2 * TYPE_DIM), jnp.float32),
    input_output_aliases={0: 0},
)


def _make_gather():
    mesh = plsc.VectorSubcoreMesh(core_axis_name="c", subcore_axis_name="s")

    @functools.partial(
        pl.kernel,
        mesh=mesh,
        compiler_params=pltpu.CompilerParams(
            use_tc_tiling_on_sc=False, needs_layout_passes=False),
        out_type=jax.ShapeDtypeStruct((_TOTAL, TYPE_DIM), jnp.float32),
        scratch_types=[
            pltpu.VMEM((_NCHUNK, _CHUNK), jnp.int32),
            pltpu.VMEM((2, _CHUNK, TYPE_DIM), jnp.float32),
            pltpu.SemaphoreType.DMA,
            pltpu.SemaphoreType.DMA,
        ],
    )
    def gather_kernel(table_hbm, hid_hbm, out_hbm, hid_v, rows_v, sem0, sem1):
        wid = lax.axis_index("s") * _NC + lax.axis_index("c")
        ibase = wid * _NCHUNK
        obase = wid * _IPW
        pltpu.sync_copy(hid_hbm.at[pl.ds(ibase, _NCHUNK)], hid_v)
        sems = [sem0, sem1]
        copies = [None, None]
        copies[0] = pltpu.async_copy(
            table_hbm.at[hid_v.at[0]], rows_v.at[0], sems[0])
        for ch in range(_NCHUNK):
            nxt = (ch + 1) % 2
            if ch + 1 < _NCHUNK:
                copies[nxt] = pltpu.async_copy(
                    table_hbm.at[hid_v.at[ch + 1]], rows_v.at[nxt], sems[nxt])
            copies[ch % 2].wait()
            pltpu.sync_copy(
                rows_v.at[ch % 2],
                out_hbm.at[pl.ds(obase + ch * _CHUNK, _CHUNK)])

    return gather_kernel


_GATHER = _make_gather()


def kernel(inputs, type_matrix):
    table_t = jnp.transpose(type_matrix)
    packed = _repack(table_t, table_t)
    tail_src = lax.slice(table_t, (0, 2 * _SPLIT), (TYPE_DIM, TYPE_NUM))
    packed = _tail(packed, tail_src)
    flat = jnp.reshape(packed, (TYPE_NUM, TYPE_DIM))

    v = jnp.reshape(inputs.astype(jnp.int32), (_TOTAL // _CHUNK, _CHUNK))
    lo = 2 * v
    mid = 2 * (v - _SPLIT) + 1
    t0 = 2 * (v - _SPLIT)
    t1 = 2 * (v - _SPLIT - 32) + 1
    hid = jnp.where(
        v < _SPLIT, lo,
        jnp.where(v < 2 * _SPLIT, mid,
                  jnp.where(v < 2 * _SPLIT + 32, t0, t1)))

    out = _GATHER(flat, hid)
    return jnp.reshape(out, (BATCH, 2 * TYPE_DIM))


# speedup vs baseline: 2.7102x; 1.0609x over previous
import functools

import jax
import jax.numpy as jnp
from jax import lax
from jax.experimental import pallas as pl
from jax.experimental.pallas import tpu as pltpu
from jax.experimental.pallas import tpu_sc as plsc

TYPE_NUM = 1000000
TYPE_DIM = 64
BATCH = 16384
_SPLIT = 499968
_HALF = TYPE_NUM // 2

_INFO = plsc.get_sparse_core_info()
_NC = _INFO.num_cores
_NS = _INFO.num_subcores
_NW = _NC * _NS
_TOTAL = BATCH * 2
_IPW = _TOTAL // _NW
_CHUNK = 128
_NCHUNK = _IPW // _CHUNK

_BLK = 16128
_GRID = _SPLIT // _BLK


def _repack_kernel(src_lo, src_hi, dst):
    dst[:, :TYPE_DIM] = jnp.transpose(src_lo[...])
    dst[:, TYPE_DIM:] = jnp.transpose(src_hi[...])


_repack = pl.pallas_call(
    _repack_kernel,
    grid=(_GRID,),
    in_specs=[
        pl.BlockSpec((TYPE_DIM, _BLK), lambda j: (0, j)),
        pl.BlockSpec((TYPE_DIM, _BLK), lambda j: (0, j + _GRID)),
    ],
    out_specs=pl.BlockSpec((_BLK, 2 * TYPE_DIM), lambda j: (j, 0)),
    out_shape=jax.ShapeDtypeStruct((_HALF, 2 * TYPE_DIM), jnp.float32),
)


def _tail_kernel(packed_ref, tsrc_ref, out_ref):
    tt = jnp.transpose(tsrc_ref[...])
    out_ref[:, :TYPE_DIM] = tt[:32, :]
    out_ref[:, TYPE_DIM:] = tt[32:, :]


_tail = pl.pallas_call(
    _tail_kernel,
    grid=(1,),
    in_specs=[
        pl.BlockSpec(memory_space=pl.ANY),
        pl.BlockSpec((TYPE_DIM, TYPE_DIM), lambda j: (0, 0)),
    ],
    out_specs=pl.BlockSpec((32, 2 * TYPE_DIM), lambda j: (_SPLIT // 32, 0)),
    out_shape=jax.ShapeDtypeStruct((_HALF, 2 * TYPE_DIM), jnp.float32),
    input_output_aliases={0: 0},
)


def _make_gather():
    mesh = plsc.VectorSubcoreMesh(core_axis_name="c", subcore_axis_name="s")

    @functools.partial(
        pl.kernel,
        mesh=mesh,
        compiler_params=pltpu.CompilerParams(
            use_tc_tiling_on_sc=False, needs_layout_passes=False),
        out_type=jax.ShapeDtypeStruct((_TOTAL, TYPE_DIM), jnp.float32),
        scratch_types=[
            pltpu.VMEM((_NCHUNK, _CHUNK), jnp.int32),
            pltpu.VMEM((2, _CHUNK, TYPE_DIM), jnp.float32),
            pltpu.SemaphoreType.DMA,
            pltpu.SemaphoreType.DMA,
        ],
    )
    def gather_kernel(table_hbm, hid_hbm, out_hbm, hid_v, rows_v, sem0, sem1):
        wid = lax.axis_index("s") * _NC + lax.axis_index("c")
        ibase = wid * _NCHUNK
        obase = wid * _IPW
        pltpu.sync_copy(hid_hbm.at[pl.ds(ibase, _NCHUNK)], hid_v)
        sems = [sem0, sem1]
        copies = [None, None]
        copies[0] = pltpu.async_copy(
            table_hbm.at[hid_v.at[0]], rows_v.at[0], sems[0])
        for ch in range(_NCHUNK):
            nxt = (ch + 1) % 2
            if ch + 1 < _NCHUNK:
                copies[nxt] = pltpu.async_copy(
                    table_hbm.at[hid_v.at[ch + 1]], rows_v.at[nxt], sems[nxt])
            copies[ch % 2].wait()
            pltpu.sync_copy(
                rows_v.at[ch % 2],
                out_hbm.at[pl.ds(obase + ch * _CHUNK, _CHUNK)])

    return gather_kernel


_GATHER = _make_gather()


def kernel(inputs, type_matrix):
    table_t = jnp.transpose(type_matrix)
    packed = _repack(table_t, table_t)
    tail_src = lax.slice(table_t, (0, 2 * _SPLIT), (TYPE_DIM, TYPE_NUM))
    packed = _tail(packed, tail_src)
    flat = jnp.reshape(packed, (TYPE_NUM, TYPE_DIM))

    v = jnp.reshape(inputs.astype(jnp.int32), (_TOTAL // _CHUNK, _CHUNK))
    lo = 2 * v
    mid = 2 * (v - _SPLIT) + 1
    t0 = 2 * (v - _SPLIT)
    t1 = 2 * (v - _SPLIT - 32) + 1
    hid = jnp.where(
        v < _SPLIT, lo,
        jnp.where(v < 2 * _SPLIT, mid,
                  jnp.where(v < 2 * _SPLIT + 32, t0, t1)))

    out = _GATHER(flat, hid)
    return jnp.reshape(out, (BATCH, 2 * TYPE_DIM))


# speedup vs baseline: 2.8412x; 1.0483x over previous
import functools

import jax
import jax.numpy as jnp
from jax import lax
from jax.experimental import pallas as pl
from jax.experimental.pallas import tpu as pltpu
from jax.experimental.pallas import tpu_sc as plsc

TYPE_NUM = 1000000
TYPE_DIM = 64
BATCH = 16384
_SPLIT = 499968
_HALF = TYPE_NUM // 2

_INFO = plsc.get_sparse_core_info()
_NC = _INFO.num_cores
_NS = _INFO.num_subcores
_NW = _NC * _NS
_TOTAL = BATCH * 2
_IPW = _TOTAL // _NW
_CHUNK = 128
_NCHUNK = _IPW // _CHUNK

_BLK = 16128
_GRID = _SPLIT // _BLK


def _repack_kernel(src_lo, src_hi, dst):
    dst[:, :TYPE_DIM] = jnp.transpose(src_lo[...])
    dst[:, TYPE_DIM:] = jnp.transpose(src_hi[...])


_repack = pl.pallas_call(
    _repack_kernel,
    grid=(_GRID,),
    in_specs=[
        pl.BlockSpec((TYPE_DIM, _BLK), lambda j: (0, j)),
        pl.BlockSpec((TYPE_DIM, _BLK), lambda j: (0, j + _GRID)),
    ],
    out_specs=pl.BlockSpec((_BLK, 2 * TYPE_DIM), lambda j: (j, 0)),
    out_shape=jax.ShapeDtypeStruct((_HALF, 2 * TYPE_DIM), jnp.float32),
)


def _tail_kernel(packed_ref, tsrc_ref, out_ref):
    tt = jnp.transpose(tsrc_ref[...])
    out_ref[:, :TYPE_DIM] = tt[:32, :]
    out_ref[:, TYPE_DIM:] = tt[32:, :]


_tail = pl.pallas_call(
    _tail_kernel,
    grid=(1,),
    in_specs=[
        pl.BlockSpec(memory_space=pl.ANY),
        pl.BlockSpec((TYPE_DIM, TYPE_DIM), lambda j: (0, 0)),
    ],
    out_specs=pl.BlockSpec((32, 2 * TYPE_DIM), lambda j: (_SPLIT // 32, 0)),
    out_shape=jax.ShapeDtypeStruct((_HALF, 2 * TYPE_DIM), jnp.float32),
    input_output_aliases={0: 0},
)


def _make_gather():
    mesh = plsc.VectorSubcoreMesh(core_axis_name="c", subcore_axis_name="s")

    @functools.partial(
        pl.kernel,
        mesh=mesh,
        compiler_params=pltpu.CompilerParams(
            use_tc_tiling_on_sc=False, needs_layout_passes=False),
        out_type=jax.ShapeDtypeStruct((_TOTAL, TYPE_DIM), jnp.float32),
        scratch_types=[
            pltpu.VMEM((_NCHUNK // 2, 2, _CHUNK), jnp.int32),
            pltpu.VMEM((_NCHUNK, _CHUNK), jnp.int32),
            pltpu.VMEM((2, _CHUNK, TYPE_DIM), jnp.float32),
            pltpu.SemaphoreType.DMA,
            pltpu.SemaphoreType.DMA,
        ],
    )
    def gather_kernel(table_hbm, vidx_hbm, out_hbm, vidx_v, hid_v, rows_v,
                      sem0, sem1):
        wid = lax.axis_index("s") * _NC + lax.axis_index("c")
        obase = wid * _IPW
        pltpu.sync_copy(vidx_hbm.at[pl.ds(wid * (_NCHUNK // 2), _NCHUNK // 2)],
                        vidx_v)
        for tt in range(_NCHUNK // 2):
            for j in range(2):
                for m in range(_CHUNK // 16):
                    v = vidx_v[tt, j, pl.ds(m * 16, 16)]
                    sel1 = jnp.where(
                        v < _SPLIT, jnp.int32(0),
                        jnp.where(v < 2 * _SPLIT + 32, jnp.int32(_SPLIT),
                                  jnp.int32(_SPLIT + 32)))
                    par = jnp.where(
                        v < _SPLIT, jnp.int32(0),
                        jnp.where(v < 2 * _SPLIT, jnp.int32(1),
                                  jnp.where(v < 2 * _SPLIT + 32, jnp.int32(0),
                                            jnp.int32(1))))
                    hid16 = 2 * (v - sel1) + par
                    ch = 2 * tt + (1 if m >= 4 else 0)
                    colbase = 2 * ((m * 16) % 64) + j
                    cols = colbase + 2 * lax.iota(jnp.int32, 16)
                    plsc.store_scatter(
                        hid_v, [jnp.full((16,), ch, jnp.int32), cols], hid16)
        sems = [sem0, sem1]
        copies = [None, None]
        copies[0] = pltpu.async_copy(
            table_hbm.at[hid_v.at[0]], rows_v.at[0], sems[0])
        for ch in range(_NCHUNK):
            nxt = (ch + 1) % 2
            if ch + 1 < _NCHUNK:
                copies[nxt] = pltpu.async_copy(
                    table_hbm.at[hid_v.at[ch + 1]], rows_v.at[nxt], sems[nxt])
            copies[ch % 2].wait()
            pltpu.sync_copy(
                rows_v.at[ch % 2],
                out_hbm.at[pl.ds(obase + ch * _CHUNK, _CHUNK)])

    return gather_kernel


_GATHER = _make_gather()


def kernel(inputs, type_matrix):
    table_t = jnp.transpose(type_matrix)
    packed = _repack(table_t, table_t)
    tail_src = lax.slice(table_t, (0, 2 * _SPLIT), (TYPE_DIM, TYPE_NUM))
    packed = _tail(packed, tail_src)
    flat = jnp.reshape(packed, (TYPE_NUM, TYPE_DIM))

    vidx = jnp.transpose(
        jnp.reshape(inputs.astype(jnp.int32), (_TOTAL // 256, _CHUNK, 2)),
        (0, 2, 1))

    out = _GATHER(flat, vidx)
    return jnp.reshape(out, (BATCH, 2 * TYPE_DIM))


# speedup vs baseline: 3.5333x; 1.2436x over previous
import functools

import jax
import jax.numpy as jnp
from jax import lax
from jax.experimental import pallas as pl
from jax.experimental.pallas import tpu as pltpu
from jax.experimental.pallas import tpu_sc as plsc

TYPE_NUM = 1000000
TYPE_DIM = 64
BATCH = 16384
_SPLIT = 499968
_HALF = TYPE_NUM // 2

_INFO = plsc.get_sparse_core_info()
_NC = _INFO.num_cores
_NS = _INFO.num_subcores
_NW = _NC * _NS
_TOTAL = BATCH * 2
_IPW = _TOTAL // _NW
_CHUNK = 128
_NCHUNK = _IPW // _CHUNK

_BLK = 16128
_GRID = _SPLIT // _BLK


def _repack_kernel(src_lo, src_hi, dst):
    x = jnp.concatenate([src_lo[...], src_hi[...]], axis=0)
    dst[...] = jnp.transpose(x)


_repack = pl.pallas_call(
    _repack_kernel,
    grid=(_GRID,),
    in_specs=[
        pl.BlockSpec((TYPE_DIM, _BLK), lambda j: (0, j)),
        pl.BlockSpec((TYPE_DIM, _BLK), lambda j: (0, j + _GRID)),
    ],
    out_specs=pl.BlockSpec((_BLK, 2 * TYPE_DIM), lambda j: (j, 0)),
    out_shape=jax.ShapeDtypeStruct((_HALF, 2 * TYPE_DIM), jnp.float32),
)


def _tail_kernel(packed_ref, tsrc_ref, out_ref):
    tt = jnp.transpose(tsrc_ref[...])
    out_ref[:, :TYPE_DIM] = tt[:32, :]
    out_ref[:, TYPE_DIM:] = tt[32:, :]


_tail = pl.pallas_call(
    _tail_kernel,
    grid=(1,),
    in_specs=[
        pl.BlockSpec(memory_space=pl.ANY),
        pl.BlockSpec((TYPE_DIM, TYPE_DIM), lambda j: (0, 0)),
    ],
    out_specs=pl.BlockSpec((32, 2 * TYPE_DIM), lambda j: (_SPLIT // 32, 0)),
    out_shape=jax.ShapeDtypeStruct((_HALF, 2 * TYPE_DIM), jnp.float32),
    input_output_aliases={0: 0},
)


def _make_gather():
    mesh = plsc.VectorSubcoreMesh(core_axis_name="c", subcore_axis_name="s")

    @functools.partial(
        pl.kernel,
        mesh=mesh,
        compiler_params=pltpu.CompilerParams(
            use_tc_tiling_on_sc=False, needs_layout_passes=False),
        out_type=jax.ShapeDtypeStruct((_TOTAL, TYPE_DIM), jnp.float32),
        scratch_types=[
            pltpu.VMEM((_NCHUNK // 2, 2, _CHUNK), jnp.int32),
            pltpu.VMEM((_NCHUNK, _CHUNK), jnp.int32),
            pltpu.VMEM((2, _CHUNK, TYPE_DIM), jnp.float32),
            pltpu.SemaphoreType.DMA,
            pltpu.SemaphoreType.DMA,
        ],
    )
    def gather_kernel(table_hbm, vidx_hbm, out_hbm, vidx_v, hid_v, rows_v,
                      sem0, sem1):
        wid = lax.axis_index("s") * _NC + lax.axis_index("c")
        obase = wid * _IPW
        pltpu.sync_copy(vidx_hbm.at[pl.ds(wid * (_NCHUNK // 2), _NCHUNK // 2)],
                        vidx_v)
        for tt in range(_NCHUNK // 2):
            for j in range(2):
                for m in range(_CHUNK // 16):
                    v = vidx_v[tt, j, pl.ds(m * 16, 16)]
                    sel1 = jnp.where(
                        v < _SPLIT, jnp.int32(0),
                        jnp.where(v < 2 * _SPLIT + 32, jnp.int32(_SPLIT),
                                  jnp.int32(_SPLIT + 32)))
                    par = jnp.where(
                        v < _SPLIT, jnp.int32(0),
                        jnp.where(v < 2 * _SPLIT, jnp.int32(1),
                                  jnp.where(v < 2 * _SPLIT + 32, jnp.int32(0),
                                            jnp.int32(1))))
                    hid16 = 2 * (v - sel1) + par
                    ch = 2 * tt + (1 if m >= 4 else 0)
                    colbase = 2 * ((m * 16) % 64) + j
                    cols = colbase + 2 * lax.iota(jnp.int32, 16)
                    plsc.store_scatter(
                        hid_v, [jnp.full((16,), ch, jnp.int32), cols], hid16)
        sems = [sem0, sem1]
        copies = [None, None]
        copies[0] = pltpu.async_copy(
            table_hbm.at[hid_v.at[0]], rows_v.at[0], sems[0])
        for ch in range(_NCHUNK):
            nxt = (ch + 1) % 2
            if ch + 1 < _NCHUNK:
                copies[nxt] = pltpu.async_copy(
                    table_hbm.at[hid_v.at[ch + 1]], rows_v.at[nxt], sems[nxt])
            copies[ch % 2].wait()
            pltpu.sync_copy(
                rows_v.at[ch % 2],
                out_hbm.at[pl.ds(obase + ch * _CHUNK, _CHUNK)])

    return gather_kernel


_GATHER = _make_gather()


def kernel(inputs, type_matrix):
    table_t = jnp.transpose(type_matrix)
    packed = _repack(table_t, table_t)
    tail_src = lax.slice(table_t, (0, 2 * _SPLIT), (TYPE_DIM, TYPE_NUM))
    packed = _tail(packed, tail_src)
    flat = jnp.reshape(packed, (TYPE_NUM, TYPE_DIM))

    vidx = jnp.transpose(
        jnp.reshape(inputs.astype(jnp.int32), (_TOTAL // 256, _CHUNK, 2)),
        (0, 2, 1))

    out = _GATHER(flat, vidx)
    return jnp.reshape(out, (BATCH, 2 * TYPE_DIM))
